# baseline (speedup 1.0000x reference)
import jax
import jax.numpy as jnp
from jax.experimental import pallas as pl
from jax.experimental.pallas import tpu as pltpu
from jax.experimental.pallas import tpu_sc as plsc

_C = 19
_B = 65536
_EPS = 1e-30
_NCORE = 2
_NSUB = 16
_NU = _NCORE * _NSUB


def _owloss_tc_kernel(g_ref, x_ref, mav_ref, out_ref, wb_ref, acc):
    i = pl.program_id(0)
    nsteps = pl.num_programs(0)

    @pl.when(i == 0)
    def _init():
        acc[...] = jnp.zeros_like(acc)
        mav = mav_ref[...]
        mns = jnp.sum(mav * mav, axis=1, keepdims=True)
        w = mav * jax.lax.rsqrt(jnp.maximum(mns, _EPS))
        wb_ref[...] = w.astype(jnp.bfloat16)

    xt = x_ref[...]
    g = g_ref[0]

    at = jax.lax.dot_general(wb_ref[...], xt, (((1,), (0,)), ((), ())),
                             preferred_element_type=jnp.float32)
    ones_row = jnp.ones((1, _C), jnp.bfloat16)
    nsq = jax.lax.dot_general(ones_row, xt * xt, (((1,), (0,)), ((), ())),
                              preferred_element_type=jnp.float32)
    rnl = jax.lax.rsqrt(jnp.maximum(nsq, _EPS))

    lbl = jax.lax.broadcasted_iota(jnp.int32, (_C, 1), 0)
    msk = lbl == g
    num = jnp.sum(jnp.where(msk, at, 0.0), axis=0, keepdims=True)
    cos = (num * rnl).astype(jnp.bfloat16)
    ohb = msk.astype(jnp.bfloat16)
    z = jax.lax.dot_general(cos, ohb, (((1,), (1,)), ((), ())),
                            preferred_element_type=jnp.float32)
    acc[...] += z

    @pl.when(i == nsteps - 1)
    def _finish():
        out_ref[...] = acc[...]


def _sc_hist(sem_gt):
    per = sem_gt.shape[0] // _NU
    x2 = sem_gt.reshape(_NU, per)
    mesh = plsc.VectorSubcoreMesh(core_axis_name="core",
                                  subcore_axis_name="subcore")

    @pl.kernel(out_type=jax.ShapeDtypeStruct((_NU, _C, 16), jnp.float32),
               mesh=mesh,
               scratch_types=[pltpu.VMEM((per,), jnp.int32),
                              pltpu.VMEM((_C, 16), jnp.float32),
                              pltpu.SemaphoreType.DMA])
    def hist_kernel(x_hbm, o_hbm, buf, acc, dsem):
        u = jax.lax.axis_index("core") * _NSUB + jax.lax.axis_index("subcore")
        pltpu.async_copy(x_hbm.at[u], buf, dsem).wait()

        def body(k, accs):
            v = buf[pl.ds(k * 16, 16)]
            return tuple(accs[l] + jnp.where(v == l, 1.0, 0.0)
                         for l in range(_C))

        accs = jax.lax.fori_loop(
            0, per // 16, body,
            tuple(jnp.zeros((16,), jnp.float32) for _ in range(_C)))
        for l in range(_C):
            acc[l, :] = accs[l]
        pltpu.async_copy(acc, o_hbm.at[u], dsem).wait()

    return hist_kernel(x2)


def _combine_kernel(cs_ref, c3_ref, pc_ref, out_ref):
    cs = cs_ref[...]
    c = jnp.sum(c3_ref[...], axis=(0, 2))[None, :]
    pc = pc_ref[...]
    lbl = jax.lax.broadcasted_iota(jnp.int32, (1, _C), 1)
    present = c > 0.0
    minl = jnp.min(jnp.where(present, lbl, _C))
    include = present & (lbl != minl) & (pc > 0.0)
    means = (c - cs) / jnp.maximum(c, 1.0)
    terms = jnp.where(include, means, 0.0)
    out_ref[...] = jnp.sum(terms, axis=(0, 1), keepdims=True).reshape(1, 1)


def kernel(logits, sem_gt, is_train, mav_table, prev_count):
    n = logits.shape[0]
    nsteps = n // _B
    xt = logits.astype(jnp.bfloat16).T
    g3 = sem_gt.reshape(nsteps, 1, _B)
    pc2 = prev_count.reshape(1, _C)
    c3 = _sc_hist(sem_gt)
    cs = pl.pallas_call(
        _owloss_tc_kernel,
        grid=(nsteps,),
        in_specs=[
            pl.BlockSpec((1, 1, _B), lambda i: (i, 0, 0)),
            pl.BlockSpec((_C, _B), lambda i: (0, i)),
            pl.BlockSpec((_C, _C), lambda i: (0, 0)),
        ],
        out_specs=pl.BlockSpec((1, _C), lambda i: (0, 0)),
        out_shape=jax.ShapeDtypeStruct((1, _C), jnp.float32),
        scratch_shapes=[
            pltpu.VMEM((_C, _C), jnp.bfloat16),
            pltpu.VMEM((1, _C), jnp.float32),
        ],
        compiler_params=pltpu.CompilerParams(
            dimension_semantics=("arbitrary",),
        ),
    )(g3, xt, mav_table)
    out = pl.pallas_call(
        _combine_kernel,
        in_specs=[
            pl.BlockSpec((1, _C), lambda: (0, 0)),
            pl.BlockSpec((_NU, _C, 16), lambda: (0, 0, 0)),
            pl.BlockSpec((1, _C), lambda: (0, 0)),
        ],
        out_specs=pl.BlockSpec((1, 1), lambda: (0, 0)),
        out_shape=jax.ShapeDtypeStruct((1, 1), jnp.float32),
    )(cs, c3, pc2)
    return jnp.reshape(out, ())


# speedup vs baseline: 1.0785x; 1.0785x over previous
import jax
import jax.numpy as jnp
from jax.experimental import pallas as pl
from jax.experimental.pallas import tpu as pltpu

_C = 19
_B = 65536
_EPS = 1e-30


def _owloss_tc_kernel(g_ref, x_ref, mav_ref, pc_ref, out_ref, wb_ref, acc):
    i = pl.program_id(0)
    nsteps = pl.num_programs(0)

    @pl.when(i == 0)
    def _init():
        acc[...] = jnp.zeros_like(acc)
        mav = mav_ref[...]
        mns = jnp.sum(mav * mav, axis=1, keepdims=True)
        w = mav * jax.lax.rsqrt(jnp.maximum(mns, _EPS))
        wb_ref[...] = w.astype(jnp.bfloat16)

    xt = x_ref[...]
    g = g_ref[0]

    at = jax.lax.dot_general(wb_ref[...], xt, (((1,), (0,)), ((), ())),
                             preferred_element_type=jnp.float32)
    ones_row = jnp.ones((1, _C), jnp.bfloat16)
    nsq = jax.lax.dot_general(ones_row, xt * xt, (((1,), (0,)), ((), ())),
                              preferred_element_type=jnp.float32)
    rnl = jax.lax.rsqrt(jnp.maximum(nsq, _EPS))

    lbl = jax.lax.broadcasted_iota(jnp.int32, (_C, 1), 0)
    msk = lbl == g
    atb = at.astype(jnp.bfloat16)
    num = jnp.sum(jnp.where(msk, atb, jnp.bfloat16(0)), axis=0, keepdims=True)
    cos = (num.astype(jnp.float32) * rnl).astype(jnp.bfloat16)
    ohb = msk.astype(jnp.bfloat16)
    cat = jnp.concatenate([cos, jnp.ones((1, _B), jnp.bfloat16)], axis=0)
    z = jax.lax.dot_general(ohb, cat, (((1,), (1,)), ((), ())),
                            preferred_element_type=jnp.float32)
    acc[...] += z

    @pl.when(i == nsteps - 1)
    def _finish():
        cs = acc[:, 0:1]
        c = acc[:, 1:2]
        pc = pc_ref[...]
        present = c > 0.0
        minl = jnp.min(jnp.where(present, lbl, _C))
        include = present & (lbl != minl) & (pc > 0.0)
        means = (c - cs) / jnp.maximum(c, 1.0)
        terms = jnp.where(include, means, 0.0)
        out_ref[...] = jnp.sum(terms, axis=(0, 1), keepdims=True).reshape(1, 1)


def kernel(logits, sem_gt, is_train, mav_table, prev_count):
    n = logits.shape[0]
    nsteps = n // _B
    xt = logits.astype(jnp.bfloat16).T
    g3 = sem_gt.reshape(nsteps, 1, _B)
    pc2 = prev_count.reshape(_C, 1)
    out = pl.pallas_call(
        _owloss_tc_kernel,
        grid=(nsteps,),
        in_specs=[
            pl.BlockSpec((1, 1, _B), lambda i: (i, 0, 0)),
            pl.BlockSpec((_C, _B), lambda i: (0, i)),
            pl.BlockSpec((_C, _C), lambda i: (0, 0)),
            pl.BlockSpec((_C, 1), lambda i: (0, 0)),
        ],
        out_specs=pl.BlockSpec((1, 1), lambda i: (0, 0)),
        out_shape=jax.ShapeDtypeStruct((1, 1), jnp.float32),
        scratch_shapes=[
            pltpu.VMEM((_C, _C), jnp.bfloat16),
            pltpu.VMEM((_C, 2), jnp.float32),
        ],
        compiler_params=pltpu.CompilerParams(
            dimension_semantics=("arbitrary",),
        ),
    )(g3, xt, mav_table, pc2)
    return jnp.reshape(out, ())


# speedup vs baseline: 1.1435x; 1.0602x over previous
import jax
import jax.numpy as jnp
from jax.experimental import pallas as pl
from jax.experimental.pallas import tpu as pltpu

_C = 19
_B = 65536
_EPS = 1e-30


def _owloss_tc_kernel(g_ref, x_ref, mav_ref, pc_ref, out_ref, wb_ref, acc):
    i = pl.program_id(0)
    nsteps = pl.num_programs(0)

    @pl.when(i == 0)
    def _init():
        acc[...] = jnp.zeros_like(acc)
        mav = mav_ref[...]
        mns = jnp.sum(mav * mav, axis=1, keepdims=True)
        w = mav * jax.lax.rsqrt(jnp.maximum(mns, _EPS))
        wb_ref[...] = w.astype(jnp.bfloat16)

    xt = x_ref[...]
    g = g_ref[0]

    at = jax.lax.dot_general(wb_ref[...], xt, (((1,), (0,)), ((), ())),
                             preferred_element_type=jnp.float32)
    ones_row = jnp.ones((1, _C), jnp.bfloat16)
    nsq = jax.lax.dot_general(ones_row, xt * xt, (((1,), (0,)), ((), ())),
                              preferred_element_type=jnp.float32)
    rnl = jax.lax.rsqrt(jnp.maximum(nsq, _EPS))

    lbl = jax.lax.broadcasted_iota(jnp.int32, (_C, 1), 0)
    msk = lbl == g
    num = jnp.sum(jnp.where(msk, at, 0.0), axis=0, keepdims=True)
    cos = (num * rnl).astype(jnp.bfloat16)
    ohb = msk.astype(jnp.bfloat16)
    cat = jnp.concatenate([cos, jnp.ones((1, _B), jnp.bfloat16)], axis=0)
    z = jax.lax.dot_general(ohb, cat, (((1,), (1,)), ((), ())),
                            preferred_element_type=jnp.float32)
    acc[...] += z

    @pl.when(i == nsteps - 1)
    def _finish():
        cs = acc[:, 0:1]
        c = acc[:, 1:2]
        pc = pc_ref[...]
        present = c > 0.0
        minl = jnp.min(jnp.where(present, lbl, _C))
        include = present & (lbl != minl) & (pc > 0.0)
        means = (c - cs) / jnp.maximum(c, 1.0)
        terms = jnp.where(include, means, 0.0)
        out_ref[...] = jnp.sum(terms, axis=(0, 1), keepdims=True).reshape(1, 1)


def kernel(logits, sem_gt, is_train, mav_table, prev_count):
    n = logits.shape[0]
    nsteps = n // _B
    xt = logits.astype(jnp.bfloat16).T
    g3 = sem_gt.reshape(nsteps, 1, _B)
    pc2 = prev_count.reshape(_C, 1)
    out = pl.pallas_call(
        _owloss_tc_kernel,
        grid=(nsteps,),
        in_specs=[
            pl.BlockSpec((1, 1, _B), lambda i: (i, 0, 0)),
            pl.BlockSpec((_C, _B), lambda i: (0, i)),
            pl.BlockSpec((_C, _C), lambda i: (0, 0)),
            pl.BlockSpec((_C, 1), lambda i: (0, 0)),
        ],
        out_specs=pl.BlockSpec((1, 1), lambda i: (0, 0)),
        out_shape=jax.ShapeDtypeStruct((1, 1), jnp.float32),
        scratch_shapes=[
            pltpu.VMEM((_C, _C), jnp.bfloat16),
            pltpu.VMEM((_C, 2), jnp.float32),
        ],
        compiler_params=pltpu.CompilerParams(
            dimension_semantics=("arbitrary",),
        ),
    )(g3, xt, mav_table, pc2)
    return jnp.reshape(out, ())
